# baseline (speedup 1.0000x reference)
import functools

import jax
import jax.numpy as jnp
from jax import lax
from jax.experimental import pallas as pl
from jax.experimental.pallas import tpu as pltpu
from jax.experimental.pallas import tpu_sc as plsc

_CLASS_NUM = 100000
_CHANNEL = 128
_SIZE = 4
_BATCH = 1024
_C = 20
_PAIRS = _BATCH * _C
_XROWS = _C * _SIZE
_NBUF = 4


def _make_sc_select():
    info = plsc.get_sparse_core_info()
    nw = info.num_cores * info.num_subcores
    rows_per_w = _PAIRS // nw
    b_per_w = _BATCH // nw
    mesh = plsc.VectorSubcoreMesh(core_axis_name="c", subcore_axis_name="s")

    @functools.partial(
        pl.kernel,
        mesh=mesh,
        out_type=jax.ShapeDtypeStruct((_BATCH, _SIZE, _XROWS, _CHANNEL),
                                      jnp.float32),
        scratch_types=[
            pltpu.VMEM((b_per_w * _XROWS,), jnp.int32),
            pltpu.VMEM((_NBUF, _XROWS, _CHANNEL), jnp.float32),
            pltpu.SemaphoreType.DMA,
            pltpu.SemaphoreType.DMA,
        ],
    )
    def sc_select(table_hbm, lab_hbm, out_hbm, idxe_v, exp_v, gsem, osem):
        wid = lax.axis_index("s") * info.num_cores + lax.axis_index("c")
        ib = wid * b_per_w
        pltpu.sync_copy(
            lab_hbm.at[pl.ds(wid * b_per_w * _XROWS, b_per_w * _XROWS)],
            idxe_v,
        )

        def gather(cc, buf):
            return pltpu.make_async_copy(
                table_hbm.at[idxe_v.at[pl.ds(cc * _XROWS, _XROWS)]],
                exp_v.at[buf],
                gsem,
            )

        def out_copies(cc, buf):
            return [
                pltpu.make_async_copy(
                    exp_v.at[buf],
                    out_hbm.at[ib + cc, s],
                    osem,
                )
                for s in range(_SIZE)
            ]

        gather(0, 0).start()

        def group_body(it, _):
            cc0 = it * _NBUF
            for b in range(_NBUF):
                cc = cc0 + b
                gather(cc, b).wait()

                drain_cc = cc - (_NBUF - 1)

                @pl.when(drain_cc >= 0)
                def _drain():
                    for cp in out_copies(drain_cc, (b + 1) % _NBUF):
                        cp.wait()

                @pl.when(cc + 1 < b_per_w)
                def _prefetch():
                    gather(cc + 1, (b + 1) % _NBUF).start()

                for cp in out_copies(cc, b):
                    cp.start()
            return 0

        lax.fori_loop(0, b_per_w // _NBUF, group_body, 0)

        for cc in range(b_per_w - (_NBUF - 1), b_per_w):
            for cp in out_copies(cc, cc % _NBUF):
                cp.wait()

    return sc_select


_SC_SELECT = _make_sc_select()


def kernel(labels, TextEmbeddings):
    table = TextEmbeddings.reshape(_CLASS_NUM, _CHANNEL)
    lab_dup = jnp.repeat(labels.reshape(_PAIRS), _SIZE)
    out4 = _SC_SELECT(table, lab_dup)
    return jnp.transpose(out4, (0, 3, 1, 2))


# speedup vs baseline: 1.5690x; 1.5690x over previous
import functools

import jax
import jax.numpy as jnp
from jax import lax
from jax.experimental import pallas as pl
from jax.experimental.pallas import tpu as pltpu
from jax.experimental.pallas import tpu_sc as plsc

_CLASS_NUM = 100000
_CHANNEL = 128
_SIZE = 4
_BATCH = 1024
_C = 20
_PAIRS = _BATCH * _C
_XROWS = _C * _SIZE


def _make_sc_select():
    info = plsc.get_sparse_core_info()
    nw = info.num_cores * info.num_subcores
    rows_per_w = _PAIRS // nw
    b_per_w = _BATCH // nw
    bc = 4
    n_chunks = b_per_w // bc
    crows = bc * _C
    mesh = plsc.VectorSubcoreMesh(core_axis_name="c", subcore_axis_name="s")

    @functools.partial(
        pl.kernel,
        mesh=mesh,
        out_type=jax.ShapeDtypeStruct((_BATCH, _SIZE, _XROWS, _CHANNEL),
                                      jnp.float32),
        scratch_types=[
            pltpu.VMEM((rows_per_w,), jnp.int32),
            pltpu.VMEM((2, crows, _CHANNEL), jnp.float32),
            pltpu.VMEM((2, bc, _XROWS, _CHANNEL), jnp.float32),
            pltpu.SemaphoreType.DMA,
            pltpu.SemaphoreType.DMA,
        ],
    )
    def sc_select(table_hbm, lab_hbm, out_hbm, idx_v, rows_v, exp_v, gsem, osem):
        wid = lax.axis_index("s") * info.num_cores + lax.axis_index("c")
        ib = wid * b_per_w
        pltpu.sync_copy(lab_hbm.at[pl.ds(wid * rows_per_w, rows_per_w)], idx_v)

        def gather(cc, buf):
            return pltpu.make_async_copy(
                table_hbm.at[idx_v.at[pl.ds(cc * crows, crows)]],
                rows_v.at[buf],
                gsem,
            )

        def out_copies(cc, buf):
            return [
                pltpu.make_async_copy(
                    exp_v.at[buf],
                    out_hbm.at[pl.ds(ib + cc * bc, bc), s],
                    osem,
                )
                for s in range(_SIZE)
            ]

        gather(0, 0).start()

        def group_body(it, _):
            cc0 = it * 2
            for b in range(2):
                cc = cc0 + b

                @pl.when(cc + 1 < n_chunks)
                def _prefetch():
                    gather(cc + 1, 1 - b).start()

                gather(cc, b).wait()

                @pl.when(cc >= 2)
                def _drain():
                    for cp in out_copies(cc - 2, b):
                        cp.wait()

                def expand_row(r, _):
                    b2 = lax.div(r, _C)
                    j = lax.rem(r, _C)
                    for l in range(_CHANNEL // 16):
                        v = rows_v[b, r, pl.ds(l * 16, 16)]
                        for t in range(_SIZE):
                            exp_v[b, b2, j * _SIZE + t, pl.ds(l * 16, 16)] = v
                    return 0

                lax.fori_loop(0, crows, expand_row, 0, unroll=2)

                for cp in out_copies(cc, b):
                    cp.start()
            return 0

        lax.fori_loop(0, n_chunks // 2, group_body, 0)

        for cc in (n_chunks - 2, n_chunks - 1):
            for cp in out_copies(cc, cc % 2):
                cp.wait()

    return sc_select


_SC_SELECT = _make_sc_select()


def kernel(labels, TextEmbeddings):
    table = TextEmbeddings.reshape(_CLASS_NUM, _CHANNEL)
    lab_flat = labels.reshape(_PAIRS)
    out4 = _SC_SELECT(table, lab_flat)
    return jnp.transpose(out4, (0, 3, 1, 2))
